# Initial kernel scaffold; baseline (speedup 1.0000x reference)
import jax
import jax.numpy as jnp
from jax.experimental import pallas as pl


def kernel(hidden_states, topk_ids, topk_weights, gate_proj, up_proj, down_proj):
    raise NotImplementedError("write your pallas kernel here")


# speedup vs baseline: 1.4120x; 1.4120x over previous
import functools

import jax
import jax.numpy as jnp
from jax import lax
from jax.experimental import pallas as pl
from jax.experimental.pallas import tpu as pltpu

_E = 8
_K = 2
_D = 768
_DFF = 2048

_BT = 512
_BF = 1024


def _ffn_body(ids_ref, w_ref, x_ref, g_ref, u_ref, d_ref, o_ref):
    e = pl.program_id(0)
    f = pl.program_id(1)
    t = pl.program_id(2)

    x = x_ref[...]
    g = g_ref[0]
    u = u_ref[0]
    d = d_ref[0]

    a = jnp.dot(x, g.T, preferred_element_type=jnp.float32)
    b = jnp.dot(x, u.T, preferred_element_type=jnp.float32)
    h = (a * jax.nn.sigmoid(a)) * b

    ids = ids_ref[...]
    w = w_ref[...]
    we = jnp.sum(jnp.where(ids == e, w, 0.0), axis=1)
    h = h * we[:, None]

    y = jnp.dot(h, d.T, preferred_element_type=jnp.float32)

    rows = pl.ds(t * _BT, _BT)

    @pl.when(jnp.logical_and(e == 0, f == 0))
    def _init():
        o_ref[rows, :] = y

    @pl.when(jnp.logical_not(jnp.logical_and(e == 0, f == 0)))
    def _acc():
        o_ref[rows, :] += y


def kernel(hidden_states, topk_ids, topk_weights, gate_proj, up_proj, down_proj):
    B, S, D = hidden_states.shape
    T = B * S
    x = hidden_states.reshape(T, D)
    ids = topk_ids.astype(jnp.int32)

    nf = _DFF // _BF
    nt = T // _BT

    out = pl.pallas_call(
        _ffn_body,
        grid=(_E, nf, nt),
        in_specs=[
            pl.BlockSpec((_BT, _K), lambda e, f, t: (t, 0)),
            pl.BlockSpec((_BT, _K), lambda e, f, t: (t, 0)),
            pl.BlockSpec((_BT, _D), lambda e, f, t: (t, 0)),
            pl.BlockSpec((1, _BF, _D), lambda e, f, t: (e, f, 0)),
            pl.BlockSpec((1, _BF, _D), lambda e, f, t: (e, f, 0)),
            pl.BlockSpec((1, _D, _BF), lambda e, f, t: (e, 0, f)),
        ],
        out_specs=pl.BlockSpec((T, _D), lambda e, f, t: (0, 0)),
        out_shape=jax.ShapeDtypeStruct((T, _D), jnp.float32),
    )(ids, topk_weights, x, gate_proj, up_proj, down_proj)

    return out.reshape(B, S, D)
